# baseline (speedup 1.0000x reference)
import jax
import jax.numpy as jnp
from jax.experimental import pallas as pl
from jax.experimental.pallas import tpu as pltpu


def _dwt_l1_body(p0_ref, p1_ref, t0_ref, t1_ref, out_ref):
    bb, cc, h, w = p0_ref.shape
    g = 64
    L = 128
    acc = jnp.zeros((g // 8, 8, L), jnp.float32)
    for p_ref, t_ref in ((p0_ref, t0_ref), (p1_ref, t1_ref)):
        for b in range(bb):
            for c in range(cc):
                for r0 in range(0, h, g):
                    for c0 in range(0, w, L):
                        e = (p_ref[b, c, r0:r0 + g, c0:c0 + L]
                             - t_ref[b, c, r0:r0 + g, c0:c0 + L])
                        e = e.reshape(g // 8, 8, L)
                        e_dn = pltpu.roll(e, 7, 1)
                        av = jnp.abs(e + e_dn)
                        aw = jnp.abs(e - e_dn)
                        avr = pltpu.roll(av, L - 1, 2)
                        awr = pltpu.roll(aw, L - 1, 2)
                        acc = acc + jnp.maximum(av, avr) + jnp.maximum(aw, awr)
    row = jax.lax.broadcasted_iota(jnp.int32, (1, 8, L), 1)
    acc = jnp.where((row & 1) == 0, acc, 0.0)
    colsum = jnp.sum(acc.reshape(g, L), axis=0, keepdims=True)
    lane = jax.lax.broadcasted_iota(jnp.int32, colsum.shape, 1)
    masked = jnp.where((lane & 1) == 0, colsum, 0.0)
    out_ref[...] = jnp.sum(masked, axis=1, keepdims=True)[None, None]


def kernel(pred, target):
    B, C, H, W = pred.shape

    BB = 2
    half0 = pl.BlockSpec((BB, C, H // 2, W), lambda i: (i, 0, 0, 0))
    half1 = pl.BlockSpec((BB, C, H // 2, W), lambda i: (i, 0, 1, 0))

    partials = pl.pallas_call(
        _dwt_l1_body,
        grid=(B // BB,),
        in_specs=[half0, half1, half0, half1],
        out_specs=pl.BlockSpec((1, 1, 1, 1), lambda i: (i, 0, 0, 0)),
        out_shape=jax.ShapeDtypeStruct((B // BB, 1, 1, 1), jnp.float32),
        compiler_params=pltpu.CompilerParams(
            dimension_semantics=("parallel",),
            vmem_limit_bytes=62 * 1024 * 1024,
        ),
    )(pred, pred, target, target)

    n = B * C * (H // 2) * (W // 2)
    return jnp.sum(partials) * (1.0 / n)


# speedup vs baseline: 1.0048x; 1.0048x over previous
import jax
import jax.numpy as jnp
from jax.experimental import pallas as pl
from jax.experimental.pallas import tpu as pltpu


def _dwt_l1_body(p0_ref, p1_ref, t0_ref, t1_ref, out_ref):
    bb, cc, h, w = p0_ref.shape
    g = 32
    L = 128
    acc = jnp.zeros((g // 8, 8, L), jnp.float32)
    for p_ref, t_ref in ((p0_ref, t0_ref), (p1_ref, t1_ref)):
        for b in range(bb):
            for c in range(cc):
                for r0 in range(0, h, g):
                    for c0 in range(0, w, L):
                        e = (p_ref[b, c, r0:r0 + g, c0:c0 + L]
                             - t_ref[b, c, r0:r0 + g, c0:c0 + L])
                        e = e.reshape(g // 8, 8, L)
                        e_dn = pltpu.roll(e, 7, 1)
                        av = jnp.abs(e + e_dn)
                        aw = jnp.abs(e - e_dn)
                        avr = pltpu.roll(av, L - 1, 2)
                        awr = pltpu.roll(aw, L - 1, 2)
                        acc = acc + jnp.maximum(av, avr) + jnp.maximum(aw, awr)
    row = jax.lax.broadcasted_iota(jnp.int32, (1, 8, L), 1)
    acc = jnp.where((row & 1) == 0, acc, 0.0)
    colsum = jnp.sum(acc.reshape(g, L), axis=0, keepdims=True)
    lane = jax.lax.broadcasted_iota(jnp.int32, colsum.shape, 1)
    masked = jnp.where((lane & 1) == 0, colsum, 0.0)
    out_ref[...] = jnp.sum(masked, axis=1, keepdims=True)[None, None]


def kernel(pred, target):
    B, C, H, W = pred.shape

    BB = 2
    half0 = pl.BlockSpec((BB, C, H // 2, W), lambda i: (i, 0, 0, 0))
    half1 = pl.BlockSpec((BB, C, H // 2, W), lambda i: (i, 0, 1, 0))

    partials = pl.pallas_call(
        _dwt_l1_body,
        grid=(B // BB,),
        in_specs=[half0, half1, half0, half1],
        out_specs=pl.BlockSpec((1, 1, 1, 1), lambda i: (i, 0, 0, 0)),
        out_shape=jax.ShapeDtypeStruct((B // BB, 1, 1, 1), jnp.float32),
        compiler_params=pltpu.CompilerParams(
            dimension_semantics=("parallel",),
            vmem_limit_bytes=62 * 1024 * 1024,
        ),
    )(pred, pred, target, target)

    n = B * C * (H // 2) * (W // 2)
    return jnp.sum(partials) * (1.0 / n)
